# baseline (speedup 1.0000x reference)
import jax
import jax.numpy as jnp
from jax.experimental import pallas as pl
from jax.experimental.pallas import tpu as pltpu

TOKENS = 32768
INP = 768
HID = 384
NF = 64
TOP_K = 8
TILE = 2048
NSTEP = TOKENS // TILE


def _body(x_ref, w1_ref, b1_ref, w2_ref, b2_ref, o_ref, s_scr):
    i = pl.program_id(0)
    buf = jax.lax.rem(i, 2)

    x = x_ref[...]
    h = jnp.maximum(
        jnp.dot(x, w1_ref[...], preferred_element_type=jnp.float32)
        + b1_ref[...],
        0.0,
    )
    logits = (
        jnp.dot(h, w2_ref[...], preferred_element_type=jnp.float32)
        + b2_ref[...]
    )
    s_scr[buf] = jax.nn.sigmoid(logits)

    s = s_scr[1 - buf]
    knocked = jnp.float32(-1.0)
    work = s
    for _ in range(TOP_K):
        m = jnp.max(work, axis=1, keepdims=True)
        work = jnp.where(work == m, knocked, work)
    o_ref[...] = jnp.where(work == knocked, s, 0.0)


def kernel(inp, W1, b1, W2, b2):
    b1r = b1.reshape(1, HID)
    b2r = b2.reshape(1, NF)
    return pl.pallas_call(
        _body,
        grid=(NSTEP + 1,),
        in_specs=[
            pl.BlockSpec((TILE, INP), lambda i: (jnp.minimum(i, NSTEP - 1), 0)),
            pl.BlockSpec((INP, HID), lambda i: (0, 0)),
            pl.BlockSpec((1, HID), lambda i: (0, 0)),
            pl.BlockSpec((HID, NF), lambda i: (0, 0)),
            pl.BlockSpec((1, NF), lambda i: (0, 0)),
        ],
        out_specs=pl.BlockSpec((TILE, NF), lambda i: (jnp.maximum(i - 1, 0), 0)),
        out_shape=jax.ShapeDtypeStruct((TOKENS, NF), jnp.float32),
        scratch_shapes=[pltpu.VMEM((2, TILE, NF), jnp.float32)],
    )(inp, W1, b1r, W2, b2r)


# speedup vs baseline: 1.1155x; 1.1155x over previous
import jax
import jax.numpy as jnp
from jax.experimental import pallas as pl

TOKENS = 32768
INP = 768
HID = 384
NF = 64
TOP_K = 8
TILE = 2048


def _fused_kernel(x_ref, w1_ref, b1_ref, w2_ref, b2_ref, o_ref):
    x = x_ref[...]
    h = jnp.maximum(
        jnp.dot(x, w1_ref[...], preferred_element_type=jnp.float32)
        + b1_ref[...],
        0.0,
    )
    logits = (
        jnp.dot(h, w2_ref[...], preferred_element_type=jnp.float32)
        + b2_ref[...]
    )
    s = jax.nn.sigmoid(logits)

    knocked = jnp.float32(-1.0)
    work = s
    for _ in range(TOP_K):
        m = jnp.max(work, axis=1, keepdims=True)
        work = jnp.where(work == m, knocked, work)
    o_ref[...] = jnp.where(work == knocked, s, 0.0)


def kernel(inp, W1, b1, W2, b2):
    b1r = b1.reshape(1, HID)
    b2r = b2.reshape(1, NF)
    grid = (TOKENS // TILE,)
    return pl.pallas_call(
        _fused_kernel,
        grid=grid,
        in_specs=[
            pl.BlockSpec((TILE, INP), lambda i: (i, 0)),
            pl.BlockSpec((INP, HID), lambda i: (0, 0)),
            pl.BlockSpec((1, HID), lambda i: (0, 0)),
            pl.BlockSpec((HID, NF), lambda i: (0, 0)),
            pl.BlockSpec((1, NF), lambda i: (0, 0)),
        ],
        out_specs=pl.BlockSpec((TILE, NF), lambda i: (i, 0)),
        out_shape=jax.ShapeDtypeStruct((TOKENS, NF), jnp.float32),
    )(inp, W1, b1r, W2, b2r)


# speedup vs baseline: 1.3796x; 1.2367x over previous
import jax
import jax.numpy as jnp
from jax.experimental import pallas as pl
from jax.experimental.pallas import tpu as pltpu

TOKENS = 32768
INP = 768
HID = 384
NF = 64
TOP_K = 8
TILE = 2048
NSTEP = TOKENS // TILE


def _body(x_ref, w1_ref, b1_ref, w2_ref, b2_ref, o_ref, s_scr):
    s = s_scr[...]
    knocked = jnp.float32(-1.0)
    work = s
    for _ in range(TOP_K):
        m = jnp.max(work, axis=1, keepdims=True)
        work = jnp.where(work == m, knocked, work)
    o_ref[...] = jnp.where(work == knocked, s, 0.0)

    x = x_ref[...]
    h = jnp.maximum(
        jnp.dot(x, w1_ref[...], preferred_element_type=jnp.float32)
        + b1_ref[...],
        0.0,
    )
    logits = (
        jnp.dot(h, w2_ref[...], preferred_element_type=jnp.float32)
        + b2_ref[...]
    )
    s_scr[...] = jax.nn.sigmoid(logits)


def kernel(inp, W1, b1, W2, b2):
    b1r = b1.reshape(1, HID)
    b2r = b2.reshape(1, NF)
    return pl.pallas_call(
        _body,
        grid=(NSTEP + 1,),
        in_specs=[
            pl.BlockSpec((TILE, INP), lambda i: (jnp.minimum(i, NSTEP - 1), 0)),
            pl.BlockSpec((INP, HID), lambda i: (0, 0)),
            pl.BlockSpec((1, HID), lambda i: (0, 0)),
            pl.BlockSpec((HID, NF), lambda i: (0, 0)),
            pl.BlockSpec((1, NF), lambda i: (0, 0)),
        ],
        out_specs=pl.BlockSpec((TILE, NF), lambda i: (jnp.maximum(i - 1, 0), 0)),
        out_shape=jax.ShapeDtypeStruct((TOKENS, NF), jnp.float32),
        scratch_shapes=[pltpu.VMEM((TILE, NF), jnp.float32)],
    )(inp, W1, b1r, W2, b2r)
